# Initial kernel scaffold; baseline (speedup 1.0000x reference)
import jax
import jax.numpy as jnp
from jax.experimental import pallas as pl


def kernel(x, edge_index, bn0_gamma, bn0_beta, gcn1_W, gcn1_b, bn1_gamma, bn1_beta, gc2_W_root, gc2_W_nbr, gc2_b, fc1_W, fc1_b, fc2_W, fc2_b):
    raise NotImplementedError("write your pallas kernel here")


# speedup vs baseline: 12.2772x; 12.2772x over previous
import functools

import jax
import jax.numpy as jnp
from jax import lax
from jax.experimental import pallas as pl
from jax.experimental.pallas import tpu as pltpu
from jax.experimental.pallas import tpu_sc as plsc

_NC = 2
_NS = 16
_CH = 80


def _pad_rows(n):
  return ((n + 1279) // 1280) * 1280


@functools.partial(jax.jit, static_argnums=(2, 3))
def _sc_degree(dst, cst, n, e):
  nw = _NC * _NS
  epw = e // nw
  iters = epw // _CH
  npad = _pad_rows(n)
  rpt = npad // _NS

  @functools.partial(
      pl.kernel,
      out_type=jax.ShapeDtypeStruct((_NC, npad, 128), jnp.float32),
      mesh=plsc.VectorSubcoreMesh(core_axis_name="c", subcore_axis_name="s"),
      scratch_types=[
          pltpu.VMEM((_CH,), jnp.int32),
          pltpu.VMEM((_CH, 128), jnp.float32),
          pltpu.VMEM_SHARED((npad, 128), jnp.float32),
      ],
  )
  def k(dst_hbm, cst_hbm, out_hbm, idx_v, ones_v, acc_sh):
    cid = lax.axis_index("c")
    sid = lax.axis_index("s")
    wid = sid * _NC + cid

    pltpu.sync_copy(cst_hbm.at[pl.ds(sid * rpt, rpt)],
                    acc_sh.at[pl.ds(sid * rpt, rpt)])
    pltpu.sync_copy(cst_hbm.at[pl.ds(npad, _CH)], ones_v)
    plsc.subcore_barrier()

    def body(i, _):
      base = wid * epw + i * _CH
      pltpu.sync_copy(dst_hbm.at[pl.ds(base, _CH)], idx_v)
      pltpu.sync_copy(ones_v, acc_sh.at[idx_v], add=True)
      return 0

    lax.fori_loop(0, iters, body, 0)
    plsc.subcore_barrier()
    pltpu.sync_copy(
        acc_sh.at[pl.ds(sid * rpt, rpt)],
        out_hbm.at[cid, pl.ds(sid * rpt, rpt)],
    )

  return k(dst, cst)


@functools.partial(jax.jit, static_argnums=(4, 5))
def _sc_edge_sum(vals, src, dst, zeros, n, e):
  nw = _NC * _NS
  epw = e // nw
  iters = epw // _CH
  npad = _pad_rows(n)
  rpt = npad // _NS

  @functools.partial(
      pl.kernel,
      out_type=jax.ShapeDtypeStruct((_NC, npad, 128), jnp.float32),
      mesh=plsc.VectorSubcoreMesh(core_axis_name="c", subcore_axis_name="s"),
      scratch_types=[
          pltpu.VMEM((_CH,), jnp.int32),
          pltpu.VMEM((_CH,), jnp.int32),
          pltpu.VMEM((_CH, 128), jnp.float32),
          pltpu.SemaphoreType.DMA,
          pltpu.VMEM_SHARED((npad, 128), jnp.float32),
      ],
  )
  def k(vals_hbm, src_hbm, dst_hbm, zeros_hbm, out_hbm, sidx, didx, rows,
        sem, acc_sh):
    cid = lax.axis_index("c")
    sid = lax.axis_index("s")
    wid = sid * _NC + cid

    pltpu.sync_copy(zeros_hbm.at[pl.ds(sid * rpt, rpt)],
                    acc_sh.at[pl.ds(sid * rpt, rpt)])
    plsc.subcore_barrier()

    def body(i, _):
      base = wid * epw + i * _CH
      pltpu.sync_copy(src_hbm.at[pl.ds(base, _CH)], sidx)
      pltpu.sync_copy(dst_hbm.at[pl.ds(base, _CH)], didx)
      pltpu.async_copy(vals_hbm.at[sidx], rows, sem).wait()
      pltpu.sync_copy(rows, acc_sh.at[didx], add=True)
      return 0

    lax.fori_loop(0, iters, body, 0)
    plsc.subcore_barrier()
    pltpu.sync_copy(
        acc_sh.at[pl.ds(sid * rpt, rpt)],
        out_hbm.at[cid, pl.ds(sid * rpt, rpt)],
    )

  return k(vals, src, dst, zeros)


def _tc_bn0_body(x_ref, g_ref, b_ref, h_ref):
  xv = x_ref[...]
  mean = jnp.mean(xv, axis=0, keepdims=True)
  var = jnp.mean((xv - mean) ** 2, axis=0, keepdims=True)
  h_ref[...] = (xv - mean) * lax.rsqrt(var + 1e-5) * g_ref[...] + b_ref[...]


def _tc_scale_body(h_ref, d_ref, m_ref, dv_ref):
  n = h_ref.shape[0]
  deg = d_ref[0, 0:n, 0:1] + d_ref[1, 0:n, 0:1] + 1.0
  dinv = lax.rsqrt(deg)
  m_ref[...] = h_ref[...] * dinv
  dv_ref[...] = jnp.broadcast_to(dinv, dv_ref.shape)


def _tc_mid_body(p_ref, m_ref, dv_ref, w1_ref, b1_ref, g1_ref, be1_ref,
                 wr_ref, wn_ref, bg_ref, z_ref, r_ref):
  n = m_ref.shape[0]
  pre = (p_ref[0, 0:n] + p_ref[1, 0:n] + m_ref[...]) * dv_ref[:, 0:1]
  h1 = jnp.dot(pre, w1_ref[...], preferred_element_type=jnp.float32)
  a = jnp.maximum(h1 + b1_ref[...], 0.0)
  mean = jnp.mean(a, axis=0, keepdims=True)
  var = jnp.mean((a - mean) ** 2, axis=0, keepdims=True)
  h2 = (a - mean) * lax.rsqrt(var + 1e-5) * g1_ref[...] + be1_ref[...]
  zz = jnp.dot(h2, wn_ref[...], preferred_element_type=jnp.float32)
  z_ref[...] = jnp.concatenate([zz, jnp.zeros_like(zz)], axis=1)
  r_ref[...] = (
      jnp.dot(h2, wr_ref[...], preferred_element_type=jnp.float32)
      + bg_ref[...]
  )


def _tc_head_body(r_ref, q_ref, w1_ref, b1_ref, w2_ref, b2_ref, o_ref):
  n, f2 = r_ref.shape
  out = jnp.maximum(
      r_ref[...] + q_ref[0, 0:n, 0:f2] + q_ref[1, 0:n, 0:f2], 0.0)
  h = jnp.maximum(
      jnp.dot(out, w1_ref[...], preferred_element_type=jnp.float32)
      + b1_ref[...], 0.0)
  o_ref[...] = (
      jnp.dot(h, w2_ref[...], preferred_element_type=jnp.float32)
      + b2_ref[...]
  )


def kernel(x, edge_index, bn0_gamma, bn0_beta, gcn1_W, gcn1_b, bn1_gamma,
           bn1_beta, gc2_W_root, gc2_W_nbr, gc2_b, fc1_W, fc1_b, fc2_W,
           fc2_b):
  n, c_in = x.shape
  e = edge_index.shape[1]
  f2 = gc2_W_root.shape[1]
  nc = fc2_W.shape[1]
  npad = _pad_rows(n)
  src = edge_index[0]
  dst = edge_index[1]

  zeros = jnp.zeros((npad, 128), jnp.float32)
  cst = jnp.concatenate([zeros, jnp.ones((_CH, 128), jnp.float32)], axis=0)

  dcnt = _sc_degree(dst, cst, n, e)

  h0 = pl.pallas_call(
      _tc_bn0_body,
      out_shape=jax.ShapeDtypeStruct((n, c_in), jnp.float32),
  )(x, bn0_gamma.reshape(1, -1), bn0_beta.reshape(1, -1))

  m, dv = pl.pallas_call(
      _tc_scale_body,
      out_shape=(
          jax.ShapeDtypeStruct((n, c_in), jnp.float32),
          jax.ShapeDtypeStruct((n, 8), jnp.float32),
      ),
  )(h0, dcnt)

  p = _sc_edge_sum(m, src, dst, zeros, n, e)

  z, r = pl.pallas_call(
      _tc_mid_body,
      out_shape=(
          jax.ShapeDtypeStruct((n, 2 * f2), jnp.float32),
          jax.ShapeDtypeStruct((n, f2), jnp.float32),
      ),
  )(p, m, dv, gcn1_W, gcn1_b.reshape(1, -1), bn1_gamma.reshape(1, -1),
    bn1_beta.reshape(1, -1), gc2_W_root, gc2_W_nbr, gc2_b.reshape(1, -1))

  q = _sc_edge_sum(z, src, dst, zeros, n, e)

  logits = pl.pallas_call(
      _tc_head_body,
      out_shape=jax.ShapeDtypeStruct((n, nc), jnp.float32),
  )(r, q, fc1_W, fc1_b.reshape(1, -1), fc2_W, fc2_b.reshape(1, -1))

  return logits
